# baseline (speedup 1.0000x reference)
import functools

import jax
import jax.numpy as jnp
from jax import lax
from jax.experimental import pallas as pl
from jax.experimental.pallas import tpu as pltpu
from jax.experimental.pallas import tpu_sc as plsc

NC = 2
NS = 16
NW = NC * NS

CHUNK = 128


@functools.partial(jax.jit, static_argnames=("n_rows", "emb_dim"))
def _gather_rows(idx2d, weight, *, n_rows, emb_dim):
    n_chunks = n_rows // CHUNK
    cpw = n_chunks // NW

    mesh = plsc.VectorSubcoreMesh(core_axis_name="c", subcore_axis_name="s")

    @functools.partial(
        pl.kernel,
        mesh=mesh,
        compiler_params=pltpu.CompilerParams(use_tc_tiling_on_sc=False),
        out_type=jax.ShapeDtypeStruct((n_rows, emb_dim), jnp.float32),
        scratch_types=[
            pltpu.VMEM((cpw, CHUNK), jnp.int32),
            pltpu.VMEM((CHUNK, emb_dim), jnp.float32),
            pltpu.SemaphoreType.DMA,
        ],
    )
    def k(idx_hbm, table_hbm, out_hbm, idx_v, rows_v, gsem):
        wid = lax.axis_index("s") * NC + lax.axis_index("c")
        row0 = wid * cpw
        pltpu.sync_copy(idx_hbm.at[wid], idx_v)

        def step(j, carry):
            pltpu.async_copy(table_hbm.at[idx_v.at[j]], rows_v, gsem).wait()
            pltpu.sync_copy(rows_v, out_hbm.at[pl.ds((row0 + j) * CHUNK, CHUNK)])
            return carry

        lax.fori_loop(0, cpw, step, 0)

    return k(idx2d, weight)


def kernel(token, weight):
    batch, hist = token.shape
    vocab, emb_dim = weight.shape
    n_rows = batch * hist
    idx2d = token.reshape(NW, n_rows // (NW * CHUNK), CHUNK)
    out = _gather_rows(idx2d, weight, n_rows=n_rows, emb_dim=emb_dim)
    return out.reshape(batch, hist, emb_dim)


# speedup vs baseline: 1.0425x; 1.0425x over previous
import functools

import jax
import jax.numpy as jnp
from jax import lax
from jax.experimental import pallas as pl
from jax.experimental.pallas import tpu as pltpu
from jax.experimental.pallas import tpu_sc as plsc

NC = 2
NS = 16
NW = NC * NS

CHUNK = 128
NBUF = 10
K = 5


@functools.partial(jax.jit, static_argnames=("n_rows", "emb_dim"))
def _gather_rows(idx2d, weight, *, n_rows, emb_dim):
    n_chunks = n_rows // CHUNK
    cpw = n_chunks // NW

    mesh = plsc.VectorSubcoreMesh(core_axis_name="c", subcore_axis_name="s")

    @functools.partial(
        pl.kernel,
        mesh=mesh,
        compiler_params=pltpu.CompilerParams(use_tc_tiling_on_sc=False),
        out_type=jax.ShapeDtypeStruct((n_rows, emb_dim), jnp.float32),
        scratch_types=[
            pltpu.VMEM((cpw, CHUNK), jnp.int32),
            pltpu.VMEM((NBUF, CHUNK, emb_dim), jnp.float32),
            pltpu.SemaphoreType.DMA,
            pltpu.SemaphoreType.DMA,
        ],
    )
    def k(idx_hbm, table_hbm, out_hbm, idx_v, rows_v, gsem, ssem):
        wid = lax.axis_index("s") * NC + lax.axis_index("c")
        row0 = wid * cpw
        pltpu.sync_copy(idx_hbm.at[wid], idx_v)

        def gather(j):
            return pltpu.make_async_copy(
                table_hbm.at[idx_v.at[j]], rows_v.at[j % NBUF], gsem)

        def store(j):
            return pltpu.make_async_copy(
                rows_v.at[j % NBUF],
                out_hbm.at[pl.ds((row0 + j) * CHUNK, CHUNK)], ssem)

        for t in range(K):
            gather(t).start()
        for j in range(cpw):
            gather(j).wait()
            store(j).start()
            f = j + K
            if f < cpw:
                if f >= NBUF:
                    store(f - NBUF).wait()
                gather(f).start()
        for j in range(cpw - NBUF, cpw):
            store(j).wait()

    return k(idx2d, weight)


def kernel(token, weight):
    batch, hist = token.shape
    vocab, emb_dim = weight.shape
    n_rows = batch * hist
    idx2d = token.reshape(NW, n_rows // (NW * CHUNK), CHUNK)
    out = _gather_rows(idx2d, weight, n_rows=n_rows, emb_dim=emb_dim)
    return out.reshape(batch, hist, emb_dim)
